# baseline (speedup 1.0000x reference)
import functools

import jax
import jax.numpy as jnp
from jax import lax
from jax.experimental import pallas as pl
from jax.experimental.pallas import tpu as pltpu
from jax.experimental.pallas import tpu_sc as plsc

D = 32
B = 16384
SEQ = 20
TEXT_V = 10000
V2 = 10008
NP = 8
NC, NS = 2, 16
NW = NC * NS
BPW = B // NW
GR = 2 * BPW
NCHUNK = GR // 16
L = 16
TCHUNK = 128


def _body(title_idx_hbm, tokT_hbm, title_tab_hbm, tab_lo_hbm, tab_hi_hbm,
          out_hbm, tab_v, tokT_v, inv_v, title_idx_v, title_rows_v, txt_blk,
          sem_title, sem_out):
    wid = lax.axis_index("s")
    ph = wid % 2

    @pl.when(ph == 0)
    def _lo():
        pltpu.sync_copy(tab_lo_hbm, tab_v)

    @pl.when(ph == 1)
    def _hi():
        pltpu.sync_copy(tab_hi_hbm, tab_v)

    iota0 = lax.iota(jnp.int32, L)
    cols0 = [jnp.full((L,), p, jnp.int32) for p in range(NP)]

    for pass_ in range(2):
      rbase = (wid // 2) * GR + pass_ * (B // 2)
      tbase = wid * BPW + pass_ * (B // 2)

      pltpu.sync_copy(title_idx_hbm.at[pl.ds(tbase, BPW)], title_idx_v)
      for j in range(BPW // TCHUNK):
          pltpu.async_copy(
              title_tab_hbm.at[title_idx_v.at[pl.ds(j * TCHUNK, TCHUNK)]],
              title_rows_v.at[pl.ds(j * TCHUNK, TCHUNK)], sem_title)

      pltpu.sync_copy(tokT_hbm.at[:, pl.ds(rbase, GR)], tokT_v)

      @pl.loop(0, NCHUNK)
      def prepass(rb):
        cnt = jnp.zeros((L,), jnp.float32)
        for t in range(SEQ):
            tv = tokT_v[t, pl.ds(rb * L, L)]
            mn = jnp.minimum(tv, 1)
            tokT_v[t, pl.ds(rb * L, L)] = tv + (1 - mn) * TEXT_V
            cnt = cnt + mn.astype(jnp.float32)
        inv_v[pl.ds(rb * L, L)] = jnp.float32(1.0) / jnp.maximum(
            cnt, jnp.float32(1e-9))

      iota = iota0
      cols = cols0
      colbase = D + L * ph

      @pl.loop(0, NCHUNK, step=2)
      def pool(rb0):
        for bslot in range(2):
            rb = rb0 + bslot

            @pl.when(rb0 > 0)
            def _drain_slot():
                pltpu.make_async_copy(
                    txt_blk.at[bslot],
                    out_hbm.at[pl.ds(0, L), pl.ds(D, L)], sem_out).wait()

            inv = inv_v[pl.ds(rb * L, L)]
            accs = [jnp.zeros((L,), jnp.float32) for _ in range(2 * NP)]
            for t in range(SEQ):
                tokv = tokT_v[t, pl.ds(rb * L, L)]
                for p in range(NP):
                    g = plsc.load_gather(tab_v, [tokv, cols[p]])
                    lo = plsc.bitcast(g << 16, jnp.float32)
                    hi = plsc.bitcast(g & jnp.int32(-65536), jnp.float32)
                    accs[2 * p] = accs[2 * p] + lo
                    accs[2 * p + 1] = accs[2 * p + 1] + hi
            for d in range(2 * NP):
                plsc.store_scatter(
                    txt_blk.at[bslot],
                    [iota, jnp.full((L,), d, jnp.int32)], accs[d] * inv)
            pltpu.async_copy(
                txt_blk.at[bslot],
                out_hbm.at[pl.ds(rbase + rb * L, L), pl.ds(colbase, L)],
                sem_out)

      for _ in range(2):
        pltpu.make_async_copy(
            txt_blk.at[0], out_hbm.at[pl.ds(0, L), pl.ds(D, L)],
            sem_out).wait()

      for j in range(BPW // TCHUNK):
        pltpu.make_async_copy(
            title_tab_hbm.at[pl.ds(0, TCHUNK)],
            title_rows_v.at[pl.ds(j * TCHUNK, TCHUNK)], sem_title).wait()
      pltpu.sync_copy(title_rows_v, out_hbm.at[pl.ds(tbase, BPW), pl.ds(0, D)])


_sc_call = pl.kernel(
    _body,
    out_type=jax.ShapeDtypeStruct((B, 2 * D), jnp.float32),
    mesh=plsc.VectorSubcoreMesh(
        core_axis_name="c", subcore_axis_name="s",
        num_cores=1, num_subcores=NS),
    scratch_types=[
        pltpu.VMEM((V2, NP), jnp.int32),
        pltpu.VMEM((SEQ, GR), jnp.int32),
        pltpu.VMEM((GR,), jnp.float32),
        pltpu.VMEM((BPW,), jnp.int32),
        pltpu.VMEM((BPW, D), jnp.float32),
        pltpu.VMEM((2, L, L), jnp.float32),
        pltpu.SemaphoreType.DMA,
        pltpu.SemaphoreType.DMA,
    ],
    compiler_params=pltpu.CompilerParams(
        use_tc_tiling_on_sc=False, needs_layout_passes=False),
)


@jax.jit
def kernel(title_idx, text_tokens, title_table, text_table):
    ti = title_idx.astype(jnp.int32)
    tokT = text_tokens.astype(jnp.int32).T
    aug = jnp.concatenate(
        [text_table, jnp.zeros((V2 - TEXT_V, D), jnp.float32)], axis=0)
    pairs = lax.bitcast_convert_type(
        aug.astype(jnp.bfloat16).reshape(V2, D // 2, 2), jnp.int32)
    return _sc_call(ti, tokT, title_table, pairs[:, :NP], pairs[:, NP:])


# speedup vs baseline: 1.1824x; 1.1824x over previous
import functools

import jax
import jax.numpy as jnp
from jax import lax
from jax.experimental import pallas as pl
from jax.experimental.pallas import tpu as pltpu
from jax.experimental.pallas import tpu_sc as plsc

D = 32
B = 16384
SEQ = 20
TEXT_V = 10000
V2 = 10008
NP = 8
NC, NS = 2, 16
NW = NC * NS
BPW = B // NW
GR = 2 * BPW
NCHUNK = GR // 16
L = 16
TCHUNK = 128


def _body(title_idx_hbm, tokT_hbm, title_tab_hbm, tab_lo_hbm, tab_hi_hbm,
          out_hbm, tab_v, tokT_v, inv_v, title_idx_v, title_rows_v, txt_blk,
          sem_title, sem_out):
    wid = lax.axis_index("s") * NC + lax.axis_index("c")
    ph = wid % 2
    rbase = (wid // 2) * GR
    tbase = wid * BPW

    @pl.when(ph == 0)
    def _lo():
        pltpu.sync_copy(tab_lo_hbm, tab_v)

    @pl.when(ph == 1)
    def _hi():
        pltpu.sync_copy(tab_hi_hbm, tab_v)

    pltpu.sync_copy(title_idx_hbm.at[pl.ds(tbase, BPW)], title_idx_v)
    for j in range(BPW // TCHUNK):
        pltpu.async_copy(
            title_tab_hbm.at[title_idx_v.at[pl.ds(j * TCHUNK, TCHUNK)]],
            title_rows_v.at[pl.ds(j * TCHUNK, TCHUNK)], sem_title)

    pltpu.sync_copy(tokT_hbm.at[:, pl.ds(rbase, GR)], tokT_v)

    @pl.loop(0, NCHUNK)
    def prepass(rb):
        cnt = jnp.zeros((L,), jnp.float32)
        for t in range(SEQ):
            tv = tokT_v[t, pl.ds(rb * L, L)]
            mn = jnp.minimum(tv, 1)
            tokT_v[t, pl.ds(rb * L, L)] = tv + (1 - mn) * TEXT_V
            cnt = cnt + mn.astype(jnp.float32)
        inv_v[pl.ds(rb * L, L)] = jnp.float32(1.0) / jnp.maximum(
            cnt, jnp.float32(1e-9))

    iota = lax.iota(jnp.int32, L)
    cols = [jnp.full((L,), p, jnp.int32) for p in range(NP)]
    colbase = D + L * ph

    @pl.loop(0, NCHUNK, step=2)
    def pool(rb0):
        for bslot in range(2):
            rb = rb0 + bslot

            @pl.when(rb0 > 0)
            def _drain_slot():
                pltpu.make_async_copy(
                    txt_blk.at[bslot],
                    out_hbm.at[pl.ds(0, L), pl.ds(D, L)], sem_out).wait()

            inv = inv_v[pl.ds(rb * L, L)]
            accs = [jnp.zeros((L,), jnp.float32) for _ in range(2 * NP)]
            for t in range(SEQ):
                tokv = tokT_v[t, pl.ds(rb * L, L)]
                for p in range(NP):
                    g = plsc.load_gather(tab_v, [tokv, cols[p]])
                    lo = plsc.bitcast(g << 16, jnp.float32)
                    hi = plsc.bitcast(g & jnp.int32(-65536), jnp.float32)
                    accs[2 * p] = accs[2 * p] + lo
                    accs[2 * p + 1] = accs[2 * p + 1] + hi
            for d in range(2 * NP):
                plsc.store_scatter(
                    txt_blk.at[bslot],
                    [iota, jnp.full((L,), d, jnp.int32)], accs[d] * inv)
            pltpu.async_copy(
                txt_blk.at[bslot],
                out_hbm.at[pl.ds(rbase + rb * L, L), pl.ds(colbase, L)],
                sem_out)

    for _ in range(2):
        pltpu.make_async_copy(
            txt_blk.at[0], out_hbm.at[pl.ds(0, L), pl.ds(D, L)],
            sem_out).wait()

    for j in range(BPW // TCHUNK):
        pltpu.make_async_copy(
            title_tab_hbm.at[pl.ds(0, TCHUNK)],
            title_rows_v.at[pl.ds(j * TCHUNK, TCHUNK)], sem_title).wait()
    pltpu.sync_copy(title_rows_v, out_hbm.at[pl.ds(tbase, BPW), pl.ds(0, D)])


_sc_call = pl.kernel(
    _body,
    out_type=jax.ShapeDtypeStruct((B, 2 * D), jnp.float32),
    mesh=plsc.VectorSubcoreMesh(
        core_axis_name="c", subcore_axis_name="s",
        num_cores=NC, num_subcores=NS),
    scratch_types=[
        pltpu.VMEM((V2, NP), jnp.int32),
        pltpu.VMEM((SEQ, GR), jnp.int32),
        pltpu.VMEM((GR,), jnp.float32),
        pltpu.VMEM((BPW,), jnp.int32),
        pltpu.VMEM((BPW, D), jnp.float32),
        pltpu.VMEM((2, L, L), jnp.float32),
        pltpu.SemaphoreType.DMA,
        pltpu.SemaphoreType.DMA,
    ],
    compiler_params=pltpu.CompilerParams(
        use_tc_tiling_on_sc=False, needs_layout_passes=False),
)


@jax.jit
def kernel(title_idx, text_tokens, title_table, text_table):
    ti = title_idx.astype(jnp.int32)
    tokT = text_tokens.astype(jnp.int32).T
    aug = jnp.concatenate(
        [text_table, jnp.zeros((V2 - TEXT_V, D), jnp.float32)], axis=0)
    pairs = lax.bitcast_convert_type(
        aug.astype(jnp.bfloat16).reshape(V2, D // 2, 2), jnp.int32)
    return _sc_call(ti, tokT, title_table, pairs[:, :NP], pairs[:, NP:])
